# baseline (speedup 1.0000x reference)
import functools

import jax
import jax.numpy as jnp
from jax import lax
from jax.experimental import pallas as pl
from jax.experimental.pallas import tpu as pltpu
from jax.experimental.pallas import tpu_sc as plsc

D_MODEL = 64
N_TOKENS = 4096 * 200
NUM_WORKERS = 32
B_PER_W = N_TOKENS // NUM_WORKERS
CHUNK = 512
N_CHUNKS = B_PER_W // CHUNK

_mesh = plsc.VectorSubcoreMesh(core_axis_name="c", subcore_axis_name="s")


@functools.partial(
    pl.kernel,
    mesh=_mesh,
    out_type=jax.ShapeDtypeStruct((N_TOKENS, D_MODEL), jnp.float32),
    scratch_types=[
        pltpu.VMEM((CHUNK,), jnp.int32),
        pltpu.VMEM((CHUNK, D_MODEL), jnp.float32),
        pltpu.SemaphoreType.DMA,
    ],
    compiler_params=pltpu.CompilerParams(use_tc_tiling_on_sc=False),
)
def _gather(table_hbm, idx_hbm, out_hbm, idx_v, rows_v, sem):
    n_cores = 2
    wid = lax.axis_index("s") * n_cores + lax.axis_index("c")
    base = wid * B_PER_W

    def body(i, carry):
        start = base + i * CHUNK
        pltpu.sync_copy(idx_hbm.at[pl.ds(start, CHUNK)], idx_v)
        pltpu.async_copy(table_hbm.at[idx_v], rows_v, sem).wait()
        pltpu.sync_copy(rows_v, out_hbm.at[pl.ds(start, CHUNK)])
        return carry

    lax.fori_loop(0, N_CHUNKS, body, 0)


def kernel(x, table):
    idx = x.reshape(-1).astype(jnp.int32)
    out = _gather(table, idx)
    return out.reshape(x.shape + (D_MODEL,))


# speedup vs baseline: 1.0435x; 1.0435x over previous
import functools

import jax
import jax.numpy as jnp
from jax import lax
from jax.experimental import pallas as pl
from jax.experimental.pallas import tpu as pltpu
from jax.experimental.pallas import tpu_sc as plsc

D_MODEL = 64
N_TOKENS = 4096 * 200
NUM_WORKERS = 32
B_PER_W = N_TOKENS // NUM_WORKERS
CHUNK = 512
N_CHUNKS = B_PER_W // CHUNK

_mesh = plsc.VectorSubcoreMesh(core_axis_name="c", subcore_axis_name="s")


@functools.partial(
    pl.kernel,
    mesh=_mesh,
    out_type=jax.ShapeDtypeStruct((N_TOKENS, D_MODEL), jnp.float32),
    scratch_types=[
        pltpu.VMEM((N_CHUNKS, CHUNK), jnp.int32),
        pltpu.VMEM((2, CHUNK, D_MODEL), jnp.float32),
        pltpu.SemaphoreType.DMA,
        pltpu.SemaphoreType.DMA,
    ],
    compiler_params=pltpu.CompilerParams(use_tc_tiling_on_sc=False),
)
def _gather(table_hbm, idx_hbm, out_hbm, idx_v, rows_v, gsem, wsem):
    n_cores = 2
    wid = lax.axis_index("s") * n_cores + lax.axis_index("c")
    base = wid * B_PER_W

    pltpu.sync_copy(idx_hbm.at[wid], idx_v)

    def g_dma(i, s):
        return pltpu.make_async_copy(
            table_hbm.at[idx_v.at[i]], rows_v.at[s], gsem)

    def w_dma(i, s):
        return pltpu.make_async_copy(
            rows_v.at[s], out_hbm.at[pl.ds(base + i * CHUNK, CHUNK)], wsem)

    g_dma(0, 0).start()

    def outer(k, carry):
        i0 = k * 2
        for b in range(2):
            i = i0 + b
            s = b

            @pl.when(i >= 1)
            def _():
                w_dma(i - 1, 1 - s).wait()

            @pl.when(i + 1 < N_CHUNKS)
            def _():
                g_dma(i + 1, 1 - s).start()

            g_dma(i, s).wait()
            w_dma(i, s).start()
        return carry

    lax.fori_loop(0, N_CHUNKS // 2, outer, 0)
    w_dma(N_CHUNKS - 1, (N_CHUNKS - 1) % 2).wait()


def kernel(x, table):
    idx = x.reshape(NUM_WORKERS, N_CHUNKS, CHUNK).astype(jnp.int32)
    out = _gather(table, idx)
    return out.reshape(x.shape + (D_MODEL,))
